# baseline (speedup 1.0000x reference)
import functools

import jax
import jax.numpy as jnp
from jax import lax
from jax.experimental import pallas as pl
from jax.experimental.pallas import tpu as pltpu
from jax.experimental.pallas import tpu_sc as plsc

EPS = 1e-5

NC = 2
NS = 16
NW = NC * NS
CHUNK = 112
DS = 632


def _mesh():
    return plsc.VectorSubcoreMesh(core_axis_name="c", subcore_axis_name="s")


def _row_splits(rows_per_tile):
    sizes = []
    left = rows_per_tile
    while left > 0:
        sizes.append(min(CHUNK, left))
        left -= min(CHUNK, left)
    return tuple(sizes)


def _sc_deg(col_p, ea_p, n_pad):
    nch = col_p.shape[1]

    def body(col_hbm, ea_hbm, out_hbm, acc, col_v, ea_v, zb):
        c = lax.axis_index("c")
        s = lax.axis_index("s")
        w = s * NC + c
        pltpu.sync_copy(col_hbm.at[w], col_v)
        pltpu.sync_copy(ea_hbm.at[w], ea_v)
        z16 = jnp.zeros((16,), jnp.float32)

        def zi(i, _):
            zb[pl.ds(i * 16, 16)] = z16
            return 0

        lax.fori_loop(0, 40, zi, 0)
        base = s * DS
        pltpu.sync_copy(zb.at[pl.ds(0, DS)], acc.at[pl.ds(base, DS)])
        plsc.subcore_barrier()

        def ch(j, _):
            pltpu.sync_copy(ea_v.at[j], acc.at[col_v.at[j]], add=True)
            return 0

        lax.fori_loop(0, nch, ch, 0)
        plsc.subcore_barrier()
        pltpu.sync_copy(acc.at[pl.ds(base, DS)], zb.at[pl.ds(0, DS)])
        pltpu.sync_copy(zb.at[pl.ds(0, DS)],
                        out_hbm.at[pl.ds(c * n_pad + base, DS)])

    call = pl.kernel(
        body,
        out_type=jax.ShapeDtypeStruct((NC * n_pad,), jnp.float32),
        mesh=_mesh(),
        compiler_params=pltpu.CompilerParams(needs_layout_passes=False),
        scratch_types=[
            pltpu.VMEM_SHARED((n_pad,), jnp.float32),
            pltpu.VMEM((nch, CHUNK), jnp.int32),
            pltpu.VMEM((nch, CHUNK), jnp.float32),
            pltpu.VMEM((640,), jnp.float32),
        ],
    )
    return call(col_p, ea_p).reshape(NC, n_pad)


def _sc_norm(dis, row_p, col_p, ea_p):
    n_pad = dis.shape[0]
    nch = row_p.shape[1]

    def body(dis_hbm, row_hbm, col_hbm, ea_hbm, out_hbm,
             dis_v, row_v, col_v, ea_v, nv):
        c = lax.axis_index("c")
        s = lax.axis_index("s")
        w = s * NC + c
        pltpu.sync_copy(dis_hbm, dis_v)
        pltpu.sync_copy(row_hbm.at[w], row_v)
        pltpu.sync_copy(col_hbm.at[w], col_v)
        pltpu.sync_copy(ea_hbm.at[w], ea_v)

        def ch(j, _):
            for q in range(CHUNK // 16):
                sl = pl.ds(q * 16, 16)
                ri = row_v[j, sl]
                ci = col_v[j, sl]
                dr = plsc.load_gather(dis_v, [ri])
                dc = plsc.load_gather(dis_v, [ci])
                nv[j, sl] = dr * dc * ea_v[j, sl]
            return 0

        lax.fori_loop(0, nch, ch, 0)
        pltpu.sync_copy(nv, out_hbm.at[w])

    call = pl.kernel(
        body,
        out_type=jax.ShapeDtypeStruct((NW, nch, CHUNK), jnp.float32),
        mesh=_mesh(),
        compiler_params=pltpu.CompilerParams(needs_layout_passes=False),
        scratch_types=[
            pltpu.VMEM((n_pad,), jnp.float32),
            pltpu.VMEM((nch, CHUNK), jnp.int32),
            pltpu.VMEM((nch, CHUNK), jnp.int32),
            pltpu.VMEM((nch, CHUNK), jnp.float32),
            pltpu.VMEM((nch, CHUNK), jnp.float32),
        ],
    )
    return call(dis, row_p, col_p, ea_p)


WSZ = 8


def _sc_hop(h, row_p, col_p, norm_p):
    n, d = h.shape
    nwin = row_p.shape[1]
    rpt = DS
    n_acc = NS * rpt
    splits = _row_splits(rpt)

    def body(h_hbm, row_hbm, col_hbm, norm_hbm, out_hbm,
             acc, row_v, col_v, norm_v, rows_buf, sem, sem3):
        c = lax.axis_index("c")
        s = lax.axis_index("s")
        w = s * NC + c
        z16 = jnp.zeros((16,), jnp.float32)

        def zi(i, _):
            for q in range(d // 16):
                rows_buf[0, i, pl.ds(q * 16, 16)] = z16
            return 0

        lax.fori_loop(0, CHUNK, zi, 0)
        base = s * rpt
        off = 0
        for sz in splits:
            pltpu.sync_copy(rows_buf.at[0, pl.ds(0, sz)],
                            acc.at[pl.ds(base + off, sz)])
            off += sz
        plsc.subcore_barrier()

        def start_gather(jw, b):
            pltpu.async_copy(h_hbm.at[row_v.at[jw]], rows_buf.at[b], sem)

        def wait_gather(jw, b):
            pltpu.make_async_copy(h_hbm.at[row_v.at[jw]], rows_buf.at[b],
                                  sem).wait()

        def start_scatter(jw, b):
            pltpu.async_copy(rows_buf.at[b], acc.at[col_v.at[jw]], sem3,
                             add=True)

        def wait_scatter(jw, b):
            pltpu.make_async_copy(rows_buf.at[b], acc.at[col_v.at[jw]],
                                  sem3).wait()

        def scale(jw, b):
            rb = rows_buf.at[b]
            nrm = norm_v.at[jw]

            def eb(i, _):
                for u in range(4):
                    e = i * 4 + u
                    sv = plsc.load_gather(
                        nrm, [jnp.full((16,), e, jnp.int32)])
                    for q in range(d // 16):
                        sl = pl.ds(q * 16, 16)
                        rb[e, sl] = rb[e, sl] * sv
                return 0

            lax.fori_loop(0, CHUNK // 4, eb, 0)

        def win(k, _):
            pltpu.sync_copy(row_hbm.at[w, k], row_v)
            pltpu.sync_copy(col_hbm.at[w, k], col_v)
            pltpu.sync_copy(norm_hbm.at[w, k], norm_v)
            start_gather(0, 0)
            for jw in range(WSZ):
                b = jw % 3
                wait_gather(jw, b)
                if jw >= 2:
                    wait_scatter(jw - 2, (jw - 2) % 3)
                if jw + 1 < WSZ:
                    start_gather(jw + 1, (jw + 1) % 3)
                scale(jw, b)
                start_scatter(jw, b)
            wait_scatter(WSZ - 2, (WSZ - 2) % 3)
            wait_scatter(WSZ - 1, (WSZ - 1) % 3)
            return 0

        lax.fori_loop(0, nwin, win, 0)
        plsc.subcore_barrier()
        off = 0
        for sz in splits:
            pltpu.sync_copy(acc.at[pl.ds(base + off, sz)],
                            rows_buf.at[0, pl.ds(0, sz)])
            pltpu.sync_copy(rows_buf.at[0, pl.ds(0, sz)],
                            out_hbm.at[c, pl.ds(base + off, sz)])
            off += sz

    call = pl.kernel(
        body,
        out_type=jax.ShapeDtypeStruct((NC, n_acc, d), jnp.float32),
        mesh=_mesh(),
        compiler_params=pltpu.CompilerParams(needs_layout_passes=False),
        scratch_types=[
            pltpu.VMEM_SHARED((n_acc, d), jnp.float32),
            pltpu.VMEM((WSZ, CHUNK), jnp.int32),
            pltpu.VMEM((WSZ, CHUNK), jnp.int32),
            pltpu.VMEM((WSZ, CHUNK), jnp.float32),
            pltpu.VMEM((3, CHUNK, d), jnp.float32),
            pltpu.SemaphoreType.DMA,
            pltpu.SemaphoreType.DMA,
        ],
    )
    return call(h, row_p, col_p, norm_p)


def _tc_prep(x, degp, w0):
    n, d = x.shape
    n_pad = degp.shape[1]

    def body(x_ref, dp_ref, w0_ref, xn_ref, out0_ref, dis_ref):
        xv = x_ref[...]
        mean = jnp.mean(xv, axis=0, keepdims=True)
        xc = xv - mean
        var = jnp.mean(xc * xc, axis=0, keepdims=True)
        xn = xc * lax.rsqrt(var + EPS)
        xn_ref[...] = xn
        out0_ref[...] = jnp.dot(xn, w0_ref[...],
                                preferred_element_type=jnp.float32)
        dp = dp_ref[...]
        deg = dp[0:1, :] + dp[1:2, :]
        pos = deg > 0.0
        dis_ref[...] = jnp.where(pos, lax.rsqrt(jnp.where(pos, deg, 1.0)), 0.0)

    return pl.pallas_call(
        body,
        out_shape=[
            jax.ShapeDtypeStruct((n, d), jnp.float32),
            jax.ShapeDtypeStruct((n, d), jnp.float32),
            jax.ShapeDtypeStruct((1, n_pad), jnp.float32),
        ],
    )(x, degp, w0)


def _tc_round(p, out_acc, w):
    n, d = out_acc.shape

    def body(p_ref, oin_ref, w_ref, h_ref, oout_ref):
        h = p_ref[0, :n] + p_ref[1, :n]
        h_ref[...] = h
        oout_ref[...] = oin_ref[...] + jnp.dot(
            h, w_ref[...], preferred_element_type=jnp.float32)

    return pl.pallas_call(
        body,
        out_shape=[
            jax.ShapeDtypeStruct((n, d), jnp.float32),
            jax.ShapeDtypeStruct((n, d), jnp.float32),
        ],
    )(p, out_acc, w)


def _tc_final(p, out_acc, w3, tag_bias, m1, b1, m2, b2):
    n, d = out_acc.shape
    dout = m2.shape[1]

    def body(p_ref, oin_ref, w3_ref, tb_ref, m1_ref, b1_ref, m2_ref, b2_ref,
             y_ref):
        h = p_ref[0, :n] + p_ref[1, :n]
        o = oin_ref[...] + jnp.dot(h, w3_ref[...],
                                   preferred_element_type=jnp.float32)
        o = jnp.maximum(o + tb_ref[...], 0.0)
        t = jnp.maximum(jnp.dot(o, m1_ref[...],
                                preferred_element_type=jnp.float32)
                        + b1_ref[...], 0.0)
        y_ref[...] = jnp.dot(t, m2_ref[...],
                             preferred_element_type=jnp.float32) + b2_ref[...]

    return pl.pallas_call(
        body,
        out_shape=jax.ShapeDtypeStruct((n, dout), jnp.float32),
    )(p, out_acc, w3, tag_bias, m1, b1, m2, b2)


def kernel(x, edge_index, edge_attr, W0, W1, W2, W3, tag_bias, M1, b1, M2, b2):
    n, d = x.shape
    e = edge_index.shape[1]
    n_pad = NS * DS
    ept = ((e + NW * WSZ * CHUNK - 1) // (NW * WSZ * CHUNK)) * WSZ * CHUNK
    ep = ept * NW
    nch = ept // CHUNK
    pad = ep - e

    row = edge_index[0]
    col = edge_index[1]
    pad_idx = (jnp.arange(pad, dtype=jnp.int32) * 37) % n
    row_p = jnp.concatenate([row, pad_idx]).reshape(NW, nch, CHUNK)
    col_p = jnp.concatenate([col, pad_idx]).reshape(NW, nch, CHUNK)
    ea_p = jnp.concatenate(
        [edge_attr, jnp.zeros((pad,), jnp.float32)]).reshape(NW, nch, CHUNK)

    degp = _sc_deg(col_p, ea_p, n_pad)
    xn, out_acc, dis = _tc_prep(x, degp, W0)
    norm_p = _sc_norm(dis.reshape(n_pad), row_p, col_p, ea_p)

    nwin = nch // WSZ
    row_p4 = row_p.reshape(NW, nwin, WSZ, CHUNK)
    col_p4 = col_p.reshape(NW, nwin, WSZ, CHUNK)
    norm_p4 = norm_p.reshape(NW, nwin, WSZ, CHUNK)

    h = xn
    for w in (W1, W2):
        part = _sc_hop(h, row_p4, col_p4, norm_p4)
        h, out_acc = _tc_round(part, out_acc, w)
    part = _sc_hop(h, row_p4, col_p4, norm_p4)
    return _tc_final(part, out_acc, W3, tag_bias.reshape(1, d),
                     M1, b1.reshape(1, -1), M2, b2.reshape(1, -1))


# speedup vs baseline: 1.0454x; 1.0454x over previous
import functools

import jax
import jax.numpy as jnp
from jax import lax
from jax.experimental import pallas as pl
from jax.experimental.pallas import tpu as pltpu
from jax.experimental.pallas import tpu_sc as plsc

EPS = 1e-5

NC = 2
NS = 16
NW = NC * NS
CHUNK = 128
DS = 632


def _mesh():
    return plsc.VectorSubcoreMesh(core_axis_name="c", subcore_axis_name="s")


def _row_splits(rows_per_tile):
    sizes = []
    left = rows_per_tile
    while left > 0:
        sizes.append(min(128, left))
        left -= min(128, left)
    return tuple(sizes)


def _sc_deg(col_p, ea_p, n_pad):
    nch = col_p.shape[1]

    def body(col_hbm, ea_hbm, out_hbm, acc, col_v, ea_v, zb):
        c = lax.axis_index("c")
        s = lax.axis_index("s")
        w = s * NC + c
        pltpu.sync_copy(col_hbm.at[w], col_v)
        pltpu.sync_copy(ea_hbm.at[w], ea_v)
        z16 = jnp.zeros((16,), jnp.float32)

        def zi(i, _):
            zb[pl.ds(i * 16, 16)] = z16
            return 0

        lax.fori_loop(0, 40, zi, 0)
        base = s * DS
        pltpu.sync_copy(zb.at[pl.ds(0, DS)], acc.at[pl.ds(base, DS)])
        plsc.subcore_barrier()

        def ch(j, _):
            pltpu.sync_copy(ea_v.at[j], acc.at[col_v.at[j]], add=True)
            return 0

        lax.fori_loop(0, nch, ch, 0)
        plsc.subcore_barrier()
        pltpu.sync_copy(acc.at[pl.ds(base, DS)], zb.at[pl.ds(0, DS)])
        pltpu.sync_copy(zb.at[pl.ds(0, DS)],
                        out_hbm.at[pl.ds(c * n_pad + base, DS)])

    call = pl.kernel(
        body,
        out_type=jax.ShapeDtypeStruct((NC * n_pad,), jnp.float32),
        mesh=_mesh(),
        compiler_params=pltpu.CompilerParams(needs_layout_passes=False),
        scratch_types=[
            pltpu.VMEM_SHARED((n_pad,), jnp.float32),
            pltpu.VMEM((nch, CHUNK), jnp.int32),
            pltpu.VMEM((nch, CHUNK), jnp.float32),
            pltpu.VMEM((640,), jnp.float32),
        ],
    )
    return call(col_p, ea_p).reshape(NC, n_pad)


def _sc_norm(dis, row_p, col_p, ea_p):
    n_pad = dis.shape[0]
    nch = row_p.shape[1]

    def body(dis_hbm, row_hbm, col_hbm, ea_hbm, out_hbm,
             dis_v, row_v, col_v, ea_v, nv):
        c = lax.axis_index("c")
        s = lax.axis_index("s")
        w = s * NC + c
        pltpu.sync_copy(dis_hbm, dis_v)
        pltpu.sync_copy(row_hbm.at[w], row_v)
        pltpu.sync_copy(col_hbm.at[w], col_v)
        pltpu.sync_copy(ea_hbm.at[w], ea_v)

        def ch(j, _):
            for q in range(CHUNK // 16):
                sl = pl.ds(q * 16, 16)
                ri = row_v[j, sl]
                ci = col_v[j, sl]
                dr = plsc.load_gather(dis_v, [ri])
                dc = plsc.load_gather(dis_v, [ci])
                nv[j, sl] = dr * dc * ea_v[j, sl]
            return 0

        lax.fori_loop(0, nch, ch, 0)
        pltpu.sync_copy(nv, out_hbm.at[w])

    call = pl.kernel(
        body,
        out_type=jax.ShapeDtypeStruct((NW, nch, CHUNK), jnp.float32),
        mesh=_mesh(),
        compiler_params=pltpu.CompilerParams(needs_layout_passes=False),
        scratch_types=[
            pltpu.VMEM((n_pad,), jnp.float32),
            pltpu.VMEM((nch, CHUNK), jnp.int32),
            pltpu.VMEM((nch, CHUNK), jnp.int32),
            pltpu.VMEM((nch, CHUNK), jnp.float32),
            pltpu.VMEM((nch, CHUNK), jnp.float32),
        ],
    )
    return call(dis, row_p, col_p, ea_p)


WSZ = 16


def _sc_hop(h, row_p, col_p, norm_p):
    n, d = h.shape
    nwin = row_p.shape[1]
    rpt = DS
    n_acc = NS * rpt
    splits = _row_splits(rpt)

    def body(h_hbm, row_hbm, col_hbm, norm_hbm, out_hbm,
             acc, row_v, col_v, norm_v, rows_buf, sem):
        c = lax.axis_index("c")
        s = lax.axis_index("s")
        w = s * NC + c
        z16 = jnp.zeros((16,), jnp.float32)

        def zi(i, _):
            for q in range(d // 16):
                rows_buf[0, i, pl.ds(q * 16, 16)] = z16
            return 0

        lax.fori_loop(0, CHUNK, zi, 0)
        base = s * rpt
        off = 0
        for sz in splits:
            pltpu.sync_copy(rows_buf.at[0, pl.ds(0, sz)],
                            acc.at[pl.ds(base + off, sz)])
            off += sz
        plsc.subcore_barrier()

        def start_gather(jw, b):
            pltpu.async_copy(h_hbm.at[row_v.at[jw]], rows_buf.at[b], sem)

        def wait_gather(jw, b):
            pltpu.make_async_copy(h_hbm.at[row_v.at[jw]], rows_buf.at[b],
                                  sem).wait()

        def chunk_work(jw, b):
            rb = rows_buf.at[b]
            nrm = norm_v.at[jw]

            def eb(i, _):
                for u in range(4):
                    e = i * 4 + u
                    sv = plsc.load_gather(
                        nrm, [jnp.full((16,), e, jnp.int32)])
                    for q in range(d // 16):
                        sl = pl.ds(q * 16, 16)
                        rb[e, sl] = rb[e, sl] * sv
                return 0

            lax.fori_loop(0, CHUNK // 4, eb, 0)
            pltpu.sync_copy(rb, acc.at[col_v.at[jw]], add=True)

        def win(k, _):
            pltpu.sync_copy(row_hbm.at[w, k], row_v)
            pltpu.sync_copy(col_hbm.at[w, k], col_v)
            pltpu.sync_copy(norm_hbm.at[w, k], norm_v)
            start_gather(0, 0)

            def inner(gi, _):
                for b in range(2):
                    jw = gi * 2 + b
                    wait_gather(jw, b)
                    start_gather(jw + 1, 1 - b)
                    chunk_work(jw, b)
                return 0

            lax.fori_loop(0, WSZ // 2 - 1, inner, 0)
            wait_gather(WSZ - 2, 0)
            start_gather(WSZ - 1, 1)
            chunk_work(WSZ - 2, 0)
            wait_gather(WSZ - 1, 1)
            chunk_work(WSZ - 1, 1)
            return 0

        lax.fori_loop(0, nwin, win, 0)
        plsc.subcore_barrier()
        off = 0
        for sz in splits:
            pltpu.sync_copy(acc.at[pl.ds(base + off, sz)],
                            rows_buf.at[0, pl.ds(0, sz)])
            pltpu.sync_copy(rows_buf.at[0, pl.ds(0, sz)],
                            out_hbm.at[c, pl.ds(base + off, sz)])
            off += sz

    call = pl.kernel(
        body,
        out_type=jax.ShapeDtypeStruct((NC, n_acc, d), jnp.float32),
        mesh=_mesh(),
        compiler_params=pltpu.CompilerParams(needs_layout_passes=False),
        scratch_types=[
            pltpu.VMEM_SHARED((n_acc, d), jnp.float32),
            pltpu.VMEM((WSZ, CHUNK), jnp.int32),
            pltpu.VMEM((WSZ, CHUNK), jnp.int32),
            pltpu.VMEM((WSZ, CHUNK), jnp.float32),
            pltpu.VMEM((2, CHUNK, d), jnp.float32),
            pltpu.SemaphoreType.DMA,
        ],
    )
    return call(h, row_p, col_p, norm_p)


def _tc_prep(x, degp, w0):
    n, d = x.shape
    n_pad = degp.shape[1]

    def body(x_ref, dp_ref, w0_ref, xn_ref, out0_ref, dis_ref):
        xv = x_ref[...]
        mean = jnp.mean(xv, axis=0, keepdims=True)
        xc = xv - mean
        var = jnp.mean(xc * xc, axis=0, keepdims=True)
        xn = xc * lax.rsqrt(var + EPS)
        xn_ref[...] = xn
        out0_ref[...] = jnp.dot(xn, w0_ref[...],
                                preferred_element_type=jnp.float32)
        dp = dp_ref[...]
        deg = dp[0:1, :] + dp[1:2, :]
        pos = deg > 0.0
        dis_ref[...] = jnp.where(pos, lax.rsqrt(jnp.where(pos, deg, 1.0)), 0.0)

    return pl.pallas_call(
        body,
        out_shape=[
            jax.ShapeDtypeStruct((n, d), jnp.float32),
            jax.ShapeDtypeStruct((n, d), jnp.float32),
            jax.ShapeDtypeStruct((1, n_pad), jnp.float32),
        ],
    )(x, degp, w0)


def _tc_round(p, out_acc, w):
    n, d = out_acc.shape

    def body(p_ref, oin_ref, w_ref, h_ref, oout_ref):
        h = p_ref[0, :n] + p_ref[1, :n]
        h_ref[...] = h
        oout_ref[...] = oin_ref[...] + jnp.dot(
            h, w_ref[...], preferred_element_type=jnp.float32)

    return pl.pallas_call(
        body,
        out_shape=[
            jax.ShapeDtypeStruct((n, d), jnp.float32),
            jax.ShapeDtypeStruct((n, d), jnp.float32),
        ],
    )(p, out_acc, w)


def _tc_final(p, out_acc, w3, tag_bias, m1, b1, m2, b2):
    n, d = out_acc.shape
    dout = m2.shape[1]

    def body(p_ref, oin_ref, w3_ref, tb_ref, m1_ref, b1_ref, m2_ref, b2_ref,
             y_ref):
        h = p_ref[0, :n] + p_ref[1, :n]
        o = oin_ref[...] + jnp.dot(h, w3_ref[...],
                                   preferred_element_type=jnp.float32)
        o = jnp.maximum(o + tb_ref[...], 0.0)
        t = jnp.maximum(jnp.dot(o, m1_ref[...],
                                preferred_element_type=jnp.float32)
                        + b1_ref[...], 0.0)
        y_ref[...] = jnp.dot(t, m2_ref[...],
                             preferred_element_type=jnp.float32) + b2_ref[...]

    return pl.pallas_call(
        body,
        out_shape=jax.ShapeDtypeStruct((n, dout), jnp.float32),
    )(p, out_acc, w3, tag_bias, m1, b1, m2, b2)


def kernel(x, edge_index, edge_attr, W0, W1, W2, W3, tag_bias, M1, b1, M2, b2):
    n, d = x.shape
    e = edge_index.shape[1]
    n_pad = NS * DS
    ept = ((e + NW * WSZ * CHUNK - 1) // (NW * WSZ * CHUNK)) * WSZ * CHUNK
    ep = ept * NW
    nch = ept // CHUNK
    pad = ep - e

    row = edge_index[0]
    col = edge_index[1]
    pad_idx = (jnp.arange(pad, dtype=jnp.int32) * 37) % n
    row_p = jnp.concatenate([row, pad_idx]).reshape(NW, nch, CHUNK)
    col_p = jnp.concatenate([col, pad_idx]).reshape(NW, nch, CHUNK)
    ea_p = jnp.concatenate(
        [edge_attr, jnp.zeros((pad,), jnp.float32)]).reshape(NW, nch, CHUNK)

    degp = _sc_deg(col_p, ea_p, n_pad)
    xn, out_acc, dis = _tc_prep(x, degp, W0)
    norm_p = _sc_norm(dis.reshape(n_pad), row_p, col_p, ea_p)

    nwin = nch // WSZ
    row_p4 = row_p.reshape(NW, nwin, WSZ, CHUNK)
    col_p4 = col_p.reshape(NW, nwin, WSZ, CHUNK)
    norm_p4 = norm_p.reshape(NW, nwin, WSZ, CHUNK)

    h = xn
    for w in (W1, W2):
        part = _sc_hop(h, row_p4, col_p4, norm_p4)
        h, out_acc = _tc_round(part, out_acc, w)
    part = _sc_hop(h, row_p4, col_p4, norm_p4)
    return _tc_final(part, out_acc, W3, tag_bias.reshape(1, d),
                     M1, b1.reshape(1, -1), M2, b2.reshape(1, -1))
